# baseline (speedup 1.0000x reference)
import functools

import jax
import jax.numpy as jnp
from jax import lax
from jax.experimental import pallas as pl
from jax.experimental.pallas import tpu as pltpu, tpu_sc as plsc

_E = 8
_TOPK = 2
_T = 64
_D = 1024
_F = 4096
_BF = 1024
_NF = _F // _BF
_L = 16


def _routing_body(rt0_hbm, rt1_hbm, rw0_hbm, rw1_hbm, tw_hbm,
                  rt0_v, rt1_v, rw0_v, rw1_v, tw_v):
    wid = lax.axis_index("s") * 2 + lax.axis_index("c")

    @pl.when(wid == 0)
    def _work():
        pltpu.sync_copy(rt0_hbm, rt0_v)
        pltpu.sync_copy(rt1_hbm, rt1_v)
        pltpu.sync_copy(rw0_hbm, rw0_v)
        pltpu.sync_copy(rw1_hbm, rw1_v)
        for e in range(_E):
            for c in range(_T // _L):
                sl = pl.ds(c * _L, _L)
                v = (jnp.where(rt0_v[sl] == e, rw0_v[sl], 0.0)
                     + jnp.where(rt1_v[sl] == e, rw1_v[sl], 0.0))
                tw_v[pl.ds(e * _T + c * _L, _L)] = v
        pltpu.sync_copy(tw_v, tw_hbm)


@functools.partial(
    pl.kernel,
    mesh=plsc.VectorSubcoreMesh(core_axis_name="c", subcore_axis_name="s"),
    out_type=jax.ShapeDtypeStruct((_E * _T,), jnp.float32),
    scratch_types=[
        pltpu.VMEM((_T,), jnp.int32),
        pltpu.VMEM((_T,), jnp.int32),
        pltpu.VMEM((_T,), jnp.float32),
        pltpu.VMEM((_T,), jnp.float32),
        pltpu.VMEM((_E * _T,), jnp.float32),
    ],
)
def _routing_sc(rt0, rt1, rw0, rw1, tw, rt0_v, rt1_v, rw0_v, rw1_v, tw_v):
    _routing_body(rt0, rt1, rw0, rw1, tw, rt0_v, rt1_v, rw0_v, rw1_v, tw_v)


def _moe_body(tw_ref, x_ref, wg_ref, wu_ref, w2_ref, out_ref):
    e = pl.program_id(0)
    j = pl.program_id(1)

    @pl.when(jnp.logical_and(e == 0, j == 0))
    def _init():
        out_ref[...] = jnp.zeros_like(out_ref)

    tok_w = tw_ref[e, :][:, None]

    x = x_ref[...].astype(jnp.bfloat16)
    wg = wg_ref[0].astype(jnp.bfloat16)
    wu = wu_ref[0].astype(jnp.bfloat16)
    w2b = w2_ref[0].astype(jnp.bfloat16)

    g = jax.lax.dot_general(x, wg, (((1,), (1,)), ((), ())),
                            preferred_element_type=jnp.float32)
    u = jax.lax.dot_general(x, wu, (((1,), (1,)), ((), ())),
                            preferred_element_type=jnp.float32)
    h = (g * jax.nn.sigmoid(g)) * u * tok_w
    o = jax.lax.dot_general(h.astype(jnp.bfloat16), w2b,
                            (((1,), (1,)), ((), ())),
                            preferred_element_type=jnp.float32)
    out_ref[...] += o


def kernel(hidden_states, expert_routing_table, router_weights, w13, w2):
    rt = expert_routing_table.astype(jnp.int32)
    tw = _routing_sc(rt[:, 0], rt[:, 1],
                     router_weights[:, 0], router_weights[:, 1])
    tw = tw.reshape(_E, _T)
    return pl.pallas_call(
        _moe_body,
        grid=(_E, _NF),
        in_specs=[
            pl.BlockSpec((_E, _T), lambda e, j: (0, 0)),
            pl.BlockSpec((_T, _D), lambda e, j: (0, 0)),
            pl.BlockSpec((1, _BF, _D), lambda e, j: (e, j, 0)),
            pl.BlockSpec((1, _BF, _D), lambda e, j: (e, _NF + j, 0)),
            pl.BlockSpec((1, _D, _BF), lambda e, j: (e, 0, j)),
        ],
        out_specs=pl.BlockSpec((_T, _D), lambda e, j: (0, 0)),
        out_shape=jax.ShapeDtypeStruct((_T, _D), jnp.float32),
    )(tw, hidden_states, w13, w13, w2)


# speedup vs baseline: 1.1733x; 1.1733x over previous
import jax
import jax.numpy as jnp
from jax import lax
from jax.experimental import pallas as pl
from jax.experimental.pallas import tpu as pltpu

_E = 8
_TOPK = 2
_T = 64
_D = 1024
_F = 4096
_BF = 1024
_NF = _F // _BF


def _moe_body(rt_ref, rw_ref, x_ref, wg_ref, wu_ref, w2_ref, out_ref, tw_ref):
    i = pl.program_id(0)
    e = i // _NF

    @pl.when(i == 0)
    def _init():
        out_ref[...] = jnp.zeros_like(out_ref)
        ei = lax.broadcasted_iota(jnp.int32, (_E, _T, _TOPK), 0)
        rt3 = rt_ref[...][None]
        rw3 = rw_ref[...][None]
        tw_ref[...] = jnp.sum(jnp.where(rt3 == ei, rw3, 0.0), axis=2)

    tok_w = tw_ref[e, :][:, None]

    x = x_ref[...].astype(jnp.bfloat16)
    wg = wg_ref[0].astype(jnp.bfloat16)
    wu = wu_ref[0].astype(jnp.bfloat16)
    w2b = w2_ref[0].astype(jnp.bfloat16)

    g = jax.lax.dot_general(x, wg, (((1,), (1,)), ((), ())),
                            preferred_element_type=jnp.float32)
    u = jax.lax.dot_general(x, wu, (((1,), (1,)), ((), ())),
                            preferred_element_type=jnp.float32)
    h = (g * jax.nn.sigmoid(g)) * u * tok_w
    o = jax.lax.dot_general(h.astype(jnp.bfloat16), w2b,
                            (((1,), (1,)), ((), ())),
                            preferred_element_type=jnp.float32)
    out_ref[...] += o


def kernel(hidden_states, expert_routing_table, router_weights, w13, w2):
    rt = expert_routing_table.astype(jnp.int32)
    return pl.pallas_call(
        _moe_body,
        grid=(_E * _NF,),
        in_specs=[
            pl.BlockSpec((_T, _TOPK), lambda i: (0, 0)),
            pl.BlockSpec((_T, _TOPK), lambda i: (0, 0)),
            pl.BlockSpec((_T, _D), lambda i: (0, 0)),
            pl.BlockSpec((1, _BF, _D), lambda i: (i // _NF, i % _NF, 0)),
            pl.BlockSpec((1, _BF, _D), lambda i: (i // _NF, _NF + i % _NF, 0)),
            pl.BlockSpec((1, _D, _BF), lambda i: (i // _NF, 0, i % _NF)),
        ],
        out_specs=pl.BlockSpec((_T, _D), lambda i: (0, 0)),
        out_shape=jax.ShapeDtypeStruct((_T, _D), jnp.float32),
        scratch_shapes=[pltpu.VMEM((_E, _T), jnp.float32)],
    )(rt, router_weights, hidden_states, w13, w13, w2)
